# baseline (speedup 1.0000x reference)
import functools

import jax
import jax.numpy as jnp
from jax import lax
from jax.experimental import pallas as pl
from jax.experimental.pallas import tpu as pltpu
from jax.experimental.pallas import tpu_sc as plsc

N = 10000
D = 128
K = 16
B = 8
H = 128
L = 128
OUT = 128

M = 200
NB = N // M
NP = 10240
CW = 256
NCH = NP // CW

NW = 32
PW = 320
NPAD = NW * PW
SUB = 8
NSUB = PW // SUB
HV = H // 16


def _knn_body(xr_ref, xall_ref, br_ref, ball_ref, w1_ref, b1_ref,
              nbr_ref, a_ref, bv_ref, dist_ref):
    xr = xr_ref[...]
    w1a = w1_ref[:D, :]
    w1b = w1_ref[D:, :]
    a_ref[...] = jnp.dot(xr, w1a - w1b,
                         preferred_element_type=jnp.float32) + b1_ref[...]
    bv_ref[...] = jnp.dot(xr, w1b, preferred_element_type=jnp.float32)

    br = br_ref[0, 0, :]
    glo = jnp.min(br)
    ghi = jnp.max(br)
    inf = jnp.float32(jnp.inf)

    def mm_chunk(c, carry):
        ballc = ball_ref[c, 0, :]
        clo = jnp.min(ballc)
        chi = jnp.max(ballc)

        @pl.when(jnp.logical_and(clo <= ghi, chi >= glo))
        def _():
            xc = xall_ref[c]
            sqc = jnp.sum(xc * xc, axis=1)
            prod = lax.dot_general(xr, xc, (((1,), (1,)), ((), ())),
                                   preferred_element_type=jnp.float32)
            s = sqc[None, :] - 2.0 * prod
            s = jnp.where(br[:, None] == ballc[None, :], s, inf)
            dist_ref[c] = s

        return carry

    lax.fori_loop(0, NCH, mm_chunk, 0)

    colid0 = lax.broadcasted_iota(jnp.int32, (M, CW), 1)
    rowk = lax.broadcasted_iota(jnp.int32, (K, M), 0)

    def k_body(k, kcarry):
        prev, nbrmat = kcarry

        def scan_chunk(c, carry):
            m, idx = carry
            ballc = ball_ref[c, 0, :]
            clo = jnp.min(ballc)
            chi = jnp.max(ballc)

            def do(m, idx):
                colid = colid0 + c * CW
                d = dist_ref[c]
                d = jnp.where(colid == prev[:, None], inf, d)
                dist_ref[c] = d
                cm = jnp.min(d, axis=1)
                cidx = jnp.min(jnp.where(d == cm[:, None], colid, NP), axis=1)
                return jnp.minimum(m, cm), jnp.where(cm < m, cidx, idx)

            return lax.cond(jnp.logical_and(clo <= ghi, chi >= glo),
                            do, lambda m_, i_: (m_, i_), m, idx)

        _, idx = lax.fori_loop(
            0, NCH, scan_chunk,
            (jnp.full((M,), inf), jnp.zeros((M,), jnp.int32)))
        return idx, jnp.where(rowk == k, idx[None, :], nbrmat)

    _, nbrmat = lax.fori_loop(
        0, K, k_body,
        (jnp.full((M,), -1, jnp.int32), jnp.zeros((K, M), jnp.int32)))
    nbr_ref[0] = nbrmat


def _knn_call(x, x_pad, batch_blocked, batch_row_pad, W1, b1):
    return pl.pallas_call(
        _knn_body,
        grid=(NB,),
        in_specs=[
            pl.BlockSpec((M, D), lambda r: (r, 0)),
            pl.BlockSpec((NCH, CW, D), lambda r: (0, 0, 0)),
            pl.BlockSpec((1, 1, M), lambda r: (r, 0, 0)),
            pl.BlockSpec((NCH, 1, CW), lambda r: (0, 0, 0)),
            pl.BlockSpec((2 * D, H), lambda r: (0, 0)),
            pl.BlockSpec((1, H), lambda r: (0, 0)),
        ],
        out_specs=[
            pl.BlockSpec((1, K, M), lambda r: (r, 0, 0)),
            pl.BlockSpec((M, H), lambda r: (r, 0)),
            pl.BlockSpec((M, H), lambda r: (r, 0)),
        ],
        out_shape=[
            jax.ShapeDtypeStruct((NB, K, M), jnp.int32),
            jax.ShapeDtypeStruct((N, H), jnp.float32),
            jax.ShapeDtypeStruct((N, H), jnp.float32),
        ],
        scratch_shapes=[pltpu.VMEM((NCH, M, CW), jnp.float32)],
    )(x, x_pad, batch_blocked, batch_row_pad, W1, b1)


def _sc_body(nbr_hbm, a_hbm, bv_hbm, r_hbm, nbr_v, a_v, rows_v, r_v, sem):
    wid = lax.axis_index("s") * 2 + lax.axis_index("c")
    base = wid * PW
    pltpu.sync_copy(nbr_hbm.at[pl.ds(base * K, PW * K)], nbr_v)
    pltpu.sync_copy(a_hbm.at[pl.ds(base * H, PW * H)], a_v)

    def sub_body(s, carry):
        idx = nbr_v.at[pl.ds(s * (SUB * K), SUB * K)]
        pltpu.async_copy(bv_hbm.at[idx], rows_v, sem).wait()
        for n in range(SUB):
            arow = (s * SUB + n) * H
            for v in range(HV):
                av = a_v[pl.ds(arow + v * 16, 16)]
                acc = jnp.zeros((16,), jnp.float32)
                for k in range(K):
                    row = rows_v[n * K + k, pl.ds(v * 16, 16)]
                    acc = acc + jnp.maximum(av + row, 0.0)
                r_v[pl.ds(arow + v * 16, 16)] = acc
        return carry

    lax.fori_loop(0, NSUB, sub_body, 0)
    pltpu.sync_copy(r_v, r_hbm.at[pl.ds(base * H, PW * H)])


def _sc_call(nbr_flat, a_flat, bv):
    mesh = plsc.VectorSubcoreMesh(core_axis_name="c", subcore_axis_name="s",
                                  num_cores=2, num_subcores=16)
    return pl.kernel(
        _sc_body,
        out_type=jax.ShapeDtypeStruct((NPAD * H,), jnp.float32),
        mesh=mesh,
        scratch_types=[
            pltpu.VMEM((PW * K,), jnp.int32),
            pltpu.VMEM((PW * H,), jnp.float32),
            pltpu.VMEM((SUB * K, H), jnp.float32),
            pltpu.VMEM((PW * H,), jnp.float32),
            pltpu.SemaphoreType.DMA,
        ],
    )(nbr_flat, a_flat, bv)


def _tail_body(r_ref, batch_ref, w2_ref, b2_ref, wf_ref, bf_ref, out_ref):
    ball = batch_ref[0, :]
    bi = lax.broadcasted_iota(jnp.int32, (B, N), 0)
    oh = jnp.where(bi == ball[None, :], 1.0, 0.0)
    s = jnp.dot(oh, r_ref[...], preferred_element_type=jnp.float32)
    cnt = jnp.sum(oh, axis=1)
    pooled_pre = s / (jnp.maximum(cnt, 1.0)[:, None] * K)
    h = jnp.dot(pooled_pre, w2_ref[...],
                preferred_element_type=jnp.float32) + b2_ref[...]
    out_ref[...] = jnp.dot(h, wf_ref[...],
                           preferred_element_type=jnp.float32) + bf_ref[...]


def _tail_call(r, batch_row, W2, b2, Wf, bf):
    return pl.pallas_call(
        _tail_body,
        out_shape=jax.ShapeDtypeStruct((B, OUT), jnp.float32),
    )(r, batch_row, W2, b2, Wf, bf)


def kernel(x, batch, W1, b1, W2, b2, Wf, bf):
    batch = batch.astype(jnp.int32)
    batch_blocked = batch.reshape(NB, 1, M)
    batch_row = batch.reshape(1, N)
    x_pad = jnp.pad(x, ((0, NP - N), (0, 0))).reshape(NCH, CW, D)
    batch_row_pad = jnp.pad(batch, (0, NP - N),
                            constant_values=B).reshape(NCH, 1, CW)

    nbr, a, bv = _knn_call(x, x_pad, batch_blocked, batch_row_pad,
                           W1, b1.reshape(1, H))

    nbr_flat = nbr.transpose(0, 2, 1).reshape(N * K)
    nbr_flat = jnp.pad(nbr_flat, (0, (NPAD - N) * K))
    a_flat = jnp.pad(a.reshape(N * H), (0, (NPAD - N) * H))

    r_flat = _sc_call(nbr_flat, a_flat, bv)
    r = r_flat[:N * H].reshape(N, H)

    return _tail_call(r, batch_row, W2, b2.reshape(1, L), Wf, bf.reshape(1, OUT))


# speedup vs baseline: 2.0454x; 2.0454x over previous
import functools

import jax
import jax.numpy as jnp
from jax import lax
from jax.experimental import pallas as pl
from jax.experimental.pallas import tpu as pltpu
from jax.experimental.pallas import tpu_sc as plsc

N = 10000
D = 128
K = 16
B = 8
H = 128
L = 128
OUT = 128

M = 200
NB = N // M
NP = 10240
CW = 256
NCH = NP // CW

NW = 32
PW = 320
NPAD = NW * PW
SUB = 8
NSUB = PW // SUB
HV = H // 16


def _knn_body(xr_ref, xall_ref, br_ref, ball_ref, w1_ref, b1_ref,
              nbr_ref, a_ref, bv_ref, dist_ref, act_ref):
    xr = xr_ref[...]
    w1a = w1_ref[:D, :]
    w1b = w1_ref[D:, :]
    a_ref[...] = jnp.dot(xr, w1a - w1b,
                         preferred_element_type=jnp.float32) + b1_ref[...]
    bv_ref[...] = jnp.dot(xr, w1b, preferred_element_type=jnp.float32)

    br = br_ref[0, 0, :]
    glo = jnp.min(br)
    ghi = jnp.max(br)
    inf = jnp.float32(jnp.inf)

    def mm_chunk(c, cnt):
        ballc = ball_ref[c, 0, :]
        clo = jnp.min(ballc)
        chi = jnp.max(ballc)
        active = jnp.logical_and(clo <= ghi, chi >= glo)

        @pl.when(active)
        def _():
            xc = xall_ref[c]
            sqc = jnp.sum(xc * xc, axis=1)
            prod = lax.dot_general(xr, xc, (((1,), (1,)), ((), ())),
                                   preferred_element_type=jnp.float32)
            s = sqc[None, :] - 2.0 * prod
            s = jnp.where(br[:, None] == ballc[None, :], s, inf)
            dist_ref[c] = s
            act_ref[cnt] = c

        return cnt + active.astype(jnp.int32)

    n_act = lax.fori_loop(0, NCH, mm_chunk, 0)

    colid0 = lax.broadcasted_iota(jnp.int32, (M, CW), 1)
    rowk = lax.broadcasted_iota(jnp.int32, (K, M), 0)

    def k_body(k, kcarry):
        prev, nbrmat = kcarry

        def scan_chunk(j, carry):
            m, idx = carry
            c = act_ref[j]
            colid = colid0 + c * CW
            d = dist_ref[c]
            d = jnp.where(colid == prev[:, None], inf, d)
            dist_ref[c] = d
            cm = jnp.min(d, axis=1)
            cidx = jnp.min(jnp.where(d == cm[:, None], colid, NP), axis=1)
            return jnp.minimum(m, cm), jnp.where(cm < m, cidx, idx)

        _, idx = lax.fori_loop(
            0, n_act, scan_chunk,
            (jnp.full((M,), inf), jnp.zeros((M,), jnp.int32)))
        return idx, jnp.where(rowk == k, idx[None, :], nbrmat)

    _, nbrmat = lax.fori_loop(
        0, K, k_body,
        (jnp.full((M,), -1, jnp.int32), jnp.zeros((K, M), jnp.int32)))
    nbr_ref[0] = nbrmat


def _knn_call(x, x_pad, batch_blocked, batch_row_pad, W1, b1):
    return pl.pallas_call(
        _knn_body,
        grid=(NB,),
        in_specs=[
            pl.BlockSpec((M, D), lambda r: (r, 0)),
            pl.BlockSpec((NCH, CW, D), lambda r: (0, 0, 0)),
            pl.BlockSpec((1, 1, M), lambda r: (r, 0, 0)),
            pl.BlockSpec((NCH, 1, CW), lambda r: (0, 0, 0)),
            pl.BlockSpec((2 * D, H), lambda r: (0, 0)),
            pl.BlockSpec((1, H), lambda r: (0, 0)),
        ],
        out_specs=[
            pl.BlockSpec((1, K, M), lambda r: (r, 0, 0)),
            pl.BlockSpec((M, H), lambda r: (r, 0)),
            pl.BlockSpec((M, H), lambda r: (r, 0)),
        ],
        out_shape=[
            jax.ShapeDtypeStruct((NB, K, M), jnp.int32),
            jax.ShapeDtypeStruct((N, H), jnp.float32),
            jax.ShapeDtypeStruct((N, H), jnp.float32),
        ],
        scratch_shapes=[pltpu.VMEM((NCH, M, CW), jnp.float32),
                        pltpu.SMEM((NCH,), jnp.int32)],
    )(x, x_pad, batch_blocked, batch_row_pad, W1, b1)


def _sc_body(nbr_hbm, a_hbm, bv_hbm, r_hbm, nbr_v, a_v, rows_v, r_v, sem):
    wid = lax.axis_index("s") * 2 + lax.axis_index("c")
    base = wid * PW
    pltpu.sync_copy(nbr_hbm.at[pl.ds(base * K, PW * K)], nbr_v)
    pltpu.sync_copy(a_hbm.at[pl.ds(base * H, PW * H)], a_v)

    def sub_body(s, carry):
        idx = nbr_v.at[pl.ds(s * (SUB * K), SUB * K)]
        pltpu.async_copy(bv_hbm.at[idx], rows_v, sem).wait()
        for n in range(SUB):
            arow = (s * SUB + n) * H
            for v in range(HV):
                av = a_v[pl.ds(arow + v * 16, 16)]
                acc = jnp.zeros((16,), jnp.float32)
                for k in range(K):
                    row = rows_v[n * K + k, pl.ds(v * 16, 16)]
                    acc = acc + jnp.maximum(av + row, 0.0)
                r_v[pl.ds(arow + v * 16, 16)] = acc
        return carry

    lax.fori_loop(0, NSUB, sub_body, 0)
    pltpu.sync_copy(r_v, r_hbm.at[pl.ds(base * H, PW * H)])


def _sc_call(nbr_flat, a_flat, bv):
    mesh = plsc.VectorSubcoreMesh(core_axis_name="c", subcore_axis_name="s",
                                  num_cores=2, num_subcores=16)
    return pl.kernel(
        _sc_body,
        out_type=jax.ShapeDtypeStruct((NPAD * H,), jnp.float32),
        mesh=mesh,
        scratch_types=[
            pltpu.VMEM((PW * K,), jnp.int32),
            pltpu.VMEM((PW * H,), jnp.float32),
            pltpu.VMEM((SUB * K, H), jnp.float32),
            pltpu.VMEM((PW * H,), jnp.float32),
            pltpu.SemaphoreType.DMA,
        ],
    )(nbr_flat, a_flat, bv)


def _tail_body(r_ref, batch_ref, w2_ref, b2_ref, wf_ref, bf_ref, out_ref):
    ball = batch_ref[0, :]
    bi = lax.broadcasted_iota(jnp.int32, (B, N), 0)
    oh = jnp.where(bi == ball[None, :], 1.0, 0.0)
    s = jnp.dot(oh, r_ref[...], preferred_element_type=jnp.float32)
    cnt = jnp.sum(oh, axis=1)
    pooled_pre = s / (jnp.maximum(cnt, 1.0)[:, None] * K)
    h = jnp.dot(pooled_pre, w2_ref[...],
                preferred_element_type=jnp.float32) + b2_ref[...]
    out_ref[...] = jnp.dot(h, wf_ref[...],
                           preferred_element_type=jnp.float32) + bf_ref[...]


def _tail_call(r, batch_row, W2, b2, Wf, bf):
    return pl.pallas_call(
        _tail_body,
        out_shape=jax.ShapeDtypeStruct((B, OUT), jnp.float32),
    )(r, batch_row, W2, b2, Wf, bf)


def kernel(x, batch, W1, b1, W2, b2, Wf, bf):
    batch = batch.astype(jnp.int32)
    batch_blocked = batch.reshape(NB, 1, M)
    batch_row = batch.reshape(1, N)
    x_pad = jnp.pad(x, ((0, NP - N), (0, 0))).reshape(NCH, CW, D)
    batch_row_pad = jnp.pad(batch, (0, NP - N),
                            constant_values=B).reshape(NCH, 1, CW)

    nbr, a, bv = _knn_call(x, x_pad, batch_blocked, batch_row_pad,
                           W1, b1.reshape(1, H))

    nbr_flat = nbr.transpose(0, 2, 1).reshape(N * K)
    nbr_flat = jnp.pad(nbr_flat, (0, (NPAD - N) * K))
    a_flat = jnp.pad(a.reshape(N * H), (0, (NPAD - N) * H))

    r_flat = _sc_call(nbr_flat, a_flat, bv)
    r = r_flat[:N * H].reshape(N, H)

    return _tail_call(r, batch_row, W2, b2.reshape(1, L), Wf, bf.reshape(1, OUT))
